# Initial kernel scaffold; baseline (speedup 1.0000x reference)
import jax
import jax.numpy as jnp
from jax.experimental import pallas as pl


def kernel(seq_tensor, wx_t, wh_t, wo_t, bxh, bo):
    raise NotImplementedError("write your pallas kernel here")


# speedup vs baseline: 1.5189x; 1.5189x over previous
import functools

import jax
import jax.numpy as jnp
from jax.experimental import pallas as pl
from jax.experimental.pallas import tpu as pltpu


def _border_masks(H, W):
    masks = []
    for dh in (-1, 0, 1):
        for dw in (-1, 0, 1):
            r = jnp.arange(H) + dh
            c = jnp.arange(W) + dw
            m = (((r >= 0) & (r < H))[:, None] &
                 ((c >= 0) & (c < W))[None, :])
            masks.append(m.reshape(1, H * W))
    return jnp.stack(masks, axis=0)


def _fused_kernel(x_ref, mask_ref, wx_ref, wh_ref, wo_ref, bxh_ref, bo_ref,
                  out_ref, xb_ref, inbr_ref, xz_ref, nbr_ref,
                  *, T, C, hidden, W, HW, F, FHW):
    shifts = [dh * W + dw for dh in (-1, 0, 1) for dw in (-1, 0, 1)]

    for f in range(F):
        xb_ref[:, f * HW:(f + 1) * HW] = (
            x_ref[0, f * C:(f + 1) * C, :].astype(jnp.bfloat16))

    xv = xb_ref[...]
    for tap, s in enumerate(shifts):
        nbr = xv if s == 0 else pltpu.roll(xv, (-s) % FHW, axis=1)
        inbr_ref[tap * C:(tap + 1) * C, :] = nbr * mask_ref[tap]
    xz_ref[...] = (
        jnp.dot(wx_ref[...], inbr_ref[...], preferred_element_type=jnp.float32)
        + bxh_ref[...])

    def emit(h_bf, t):
        y = (jnp.dot(wo_ref[...], h_bf, preferred_element_type=jnp.float32)
             + bo_ref[...])
        for f in range(F):
            out_ref[0, f, t] = y[:, f * HW:(f + 1) * HW]

    h = jnp.tanh(xz_ref[...])
    emit(h.astype(jnp.bfloat16), 0)

    for t in range(1, T):
        for tap, s in enumerate(shifts):
            nbr = h if s == 0 else pltpu.roll(h, (-s) % FHW, axis=1)
            nbr_ref[tap * hidden:(tap + 1) * hidden, :] = (
                nbr.astype(jnp.bfloat16) * mask_ref[tap])
        conv = jnp.dot(wh_ref[...], nbr_ref[...],
                       preferred_element_type=jnp.float32)
        h = jnp.tanh(xz_ref[...] + conv)
        emit(h.astype(jnp.bfloat16), t)


def kernel(seq_tensor, wx_t, wh_t, wo_t, bxh, bo):
    batch, seqlen, H, W, C = seq_tensor.shape
    N, HW = batch * seqlen, H * W
    T = 5
    hidden = wh_t.shape[1]

    F = max(1, min(2, N))
    while N % F != 0:
        F -= 1
    G = N // F
    FHW = F * HW

    x = seq_tensor.reshape(G, F * C, HW)

    wx = wx_t.astype(jnp.bfloat16)
    wh = (wh_t.transpose(1, 0, 2)
          .reshape(hidden, 9 * hidden).astype(jnp.bfloat16))
    masks = jnp.tile(_border_masks(H, W), (1, 1, F)).astype(jnp.bfloat16)

    body = functools.partial(_fused_kernel, T=T, C=C, hidden=hidden,
                             W=W, HW=HW, F=F, FHW=FHW)
    y = pl.pallas_call(
        body,
        out_shape=jax.ShapeDtypeStruct((G, F, T, C, HW), jnp.float32),
        grid=(G,),
        in_specs=[
            pl.BlockSpec((1, F * C, HW), lambda g: (g, 0, 0)),
            pl.BlockSpec((9, 1, FHW), lambda g: (0, 0, 0)),
            pl.BlockSpec((hidden, 9 * C), lambda g: (0, 0)),
            pl.BlockSpec((hidden, 9 * hidden), lambda g: (0, 0)),
            pl.BlockSpec((C, hidden), lambda g: (0, 0)),
            pl.BlockSpec((hidden, 1), lambda g: (0, 0)),
            pl.BlockSpec((C, 1), lambda g: (0, 0)),
        ],
        out_specs=pl.BlockSpec((1, F, T, C, HW), lambda g: (g, 0, 0, 0, 0)),
        scratch_shapes=[
            pltpu.VMEM((C, FHW), jnp.bfloat16),
            pltpu.VMEM((9 * C, FHW), jnp.bfloat16),
            pltpu.VMEM((hidden, FHW), jnp.float32),
            pltpu.VMEM((9 * hidden, FHW), jnp.bfloat16),
        ],
        compiler_params=pltpu.CompilerParams(
            dimension_semantics=("parallel",),
            vmem_limit_bytes=40 * 1024 * 1024),
    )(x, masks, wx, wh, wo_t, bxh, bo)

    return y.reshape(N, T, C, H, W)


# speedup vs baseline: 1.9104x; 1.2578x over previous
import functools

import jax
import jax.numpy as jnp
from jax.experimental import pallas as pl
from jax.experimental.pallas import tpu as pltpu


def _border_masks(H, W):
    masks = []
    for dh in (-1, 0, 1):
        for dw in (-1, 0, 1):
            r = jnp.arange(H) + dh
            c = jnp.arange(W) + dw
            m = (((r >= 0) & (r < H))[:, None] &
                 ((c >= 0) & (c < W))[None, :])
            masks.append(m.reshape(1, H * W))
    return jnp.stack(masks, axis=0)


def _col_row_masks(H, W):
    c = jnp.arange(W)
    r = jnp.arange(H)
    ones_r = jnp.ones((H, 1))
    ones_c = jnp.ones((1, W))
    cm = jnp.stack([(ones_r * ((c - 1) >= 0)[None, :]).reshape(1, H * W),
                    (ones_r * ((c + 1) < W)[None, :]).reshape(1, H * W)], axis=0)
    rm = jnp.stack([(((r - 1) >= 0)[:, None] * ones_c).reshape(1, H * W),
                    (((r + 1) < H)[:, None] * ones_c).reshape(1, H * W)], axis=0)
    return cm, rm


def _fused_kernel(x_ref, mask_ref, cm_ref, rm_ref, wx_ref, wmid_ref, wpm_ref,
                  wo_ref, bxh_ref, bo_ref, out_ref,
                  xb_ref, inbr_ref, xz_ref, cstack_ref, u_ref,
                  *, T, C, hidden, W, HW, F, FHW):
    shifts = [dh * W + dw for dh in (-1, 0, 1) for dw in (-1, 0, 1)]

    for f in range(F):
        xb_ref[:, f * HW:(f + 1) * HW] = (
            x_ref[0, f * C:(f + 1) * C, :].astype(jnp.bfloat16))

    xv = xb_ref[...]
    for tap, s in enumerate(shifts):
        if s == 0:
            inbr_ref[tap * C:(tap + 1) * C, :] = xv
        else:
            inbr_ref[tap * C:(tap + 1) * C, :] = (
                pltpu.roll(xv, (-s) % FHW, axis=1) * mask_ref[tap])
    xz_ref[...] = (
        jnp.dot(wx_ref[...], inbr_ref[...], preferred_element_type=jnp.float32)
        + bxh_ref[...])

    def emit(h_bf, t):
        y = (jnp.dot(wo_ref[...], h_bf, preferred_element_type=jnp.float32)
             + bo_ref[...])
        for f in range(F):
            out_ref[0, f, t] = y[:, f * HW:(f + 1) * HW]

    h = jnp.tanh(xz_ref[...])
    emit(h.astype(jnp.bfloat16), 0)

    for t in range(1, T):
        cstack_ref[0:hidden, :] = (
            pltpu.roll(h, 1, axis=1) * cm_ref[0]).astype(jnp.bfloat16)
        cstack_ref[hidden:2 * hidden, :] = h.astype(jnp.bfloat16)
        cstack_ref[2 * hidden:3 * hidden, :] = (
            pltpu.roll(h, FHW - 1, axis=1) * cm_ref[1]).astype(jnp.bfloat16)
        u_ref[...] = jnp.dot(wpm_ref[...], cstack_ref[...],
                             preferred_element_type=jnp.float32)
        conv = (jnp.dot(wmid_ref[...], cstack_ref[...],
                        preferred_element_type=jnp.float32)
                + pltpu.roll(u_ref[0:hidden, :], W, axis=1) * rm_ref[0]
                + pltpu.roll(u_ref[hidden:2 * hidden, :], FHW - W,
                             axis=1) * rm_ref[1])
        h = jnp.tanh(xz_ref[...] + conv)
        emit(h.astype(jnp.bfloat16), t)


def kernel(seq_tensor, wx_t, wh_t, wo_t, bxh, bo):
    batch, seqlen, H, W, C = seq_tensor.shape
    N, HW = batch * seqlen, H * W
    T = 5
    hidden = wh_t.shape[1]

    F = max(1, min(2, N))
    while N % F != 0:
        F -= 1
    G = N // F
    FHW = F * HW

    x = seq_tensor.reshape(G, F * C, HW)

    wx = wx_t.astype(jnp.bfloat16)
    wg = (wh_t.reshape(3, 3, hidden, hidden)
          .transpose(0, 2, 1, 3)
          .reshape(3 * hidden, 3 * hidden).astype(jnp.bfloat16))
    wmid = wg[hidden:2 * hidden, :]
    wpm = jnp.concatenate([wg[0:hidden, :], wg[2 * hidden:3 * hidden, :]],
                          axis=0)
    masks = jnp.tile(_border_masks(H, W), (1, 1, F)).astype(jnp.bfloat16)
    cm, rm = _col_row_masks(H, W)
    cm = jnp.tile(cm, (1, 1, F)).astype(jnp.float32)
    rm = jnp.tile(rm, (1, 1, F)).astype(jnp.float32)

    body = functools.partial(_fused_kernel, T=T, C=C, hidden=hidden,
                             W=W, HW=HW, F=F, FHW=FHW)
    y = pl.pallas_call(
        body,
        out_shape=jax.ShapeDtypeStruct((G, F, T, C, HW), jnp.float32),
        grid=(G,),
        in_specs=[
            pl.BlockSpec((1, F * C, HW), lambda g: (g, 0, 0)),
            pl.BlockSpec((9, 1, FHW), lambda g: (0, 0, 0)),
            pl.BlockSpec((2, 1, FHW), lambda g: (0, 0, 0)),
            pl.BlockSpec((2, 1, FHW), lambda g: (0, 0, 0)),
            pl.BlockSpec((hidden, 9 * C), lambda g: (0, 0)),
            pl.BlockSpec((hidden, 3 * hidden), lambda g: (0, 0)),
            pl.BlockSpec((2 * hidden, 3 * hidden), lambda g: (0, 0)),
            pl.BlockSpec((C, hidden), lambda g: (0, 0)),
            pl.BlockSpec((hidden, 1), lambda g: (0, 0)),
            pl.BlockSpec((C, 1), lambda g: (0, 0)),
        ],
        out_specs=pl.BlockSpec((1, F, T, C, HW), lambda g: (g, 0, 0, 0, 0)),
        scratch_shapes=[
            pltpu.VMEM((C, FHW), jnp.bfloat16),
            pltpu.VMEM((9 * C, FHW), jnp.bfloat16),
            pltpu.VMEM((hidden, FHW), jnp.float32),
            pltpu.VMEM((3 * hidden, FHW), jnp.bfloat16),
            pltpu.VMEM((2 * hidden, FHW), jnp.float32),
        ],
        compiler_params=pltpu.CompilerParams(
            dimension_semantics=("parallel",),
            vmem_limit_bytes=40 * 1024 * 1024),
    )(x, masks, cm, rm, wx, wmid, wpm, wo_t, bxh, bo)

    return y.reshape(N, T, C, H, W)


# speedup vs baseline: 2.0932x; 1.0957x over previous
import functools

import jax
import jax.numpy as jnp
from jax.experimental import pallas as pl
from jax.experimental.pallas import tpu as pltpu


def _border_masks(H, W):
    masks = []
    for dh in (-1, 0, 1):
        for dw in (-1, 0, 1):
            r = jnp.arange(H) + dh
            c = jnp.arange(W) + dw
            m = (((r >= 0) & (r < H))[:, None] &
                 ((c >= 0) & (c < W))[None, :])
            masks.append(m.reshape(1, H * W))
    return jnp.stack(masks, axis=0)


def _col_row_masks(H, W):
    c = jnp.arange(W)
    r = jnp.arange(H)
    ones_r = jnp.ones((H, 1))
    ones_c = jnp.ones((1, W))
    cm = jnp.stack([(ones_r * ((c - 1) >= 0)[None, :]).reshape(1, H * W),
                    (ones_r * ((c + 1) < W)[None, :]).reshape(1, H * W)], axis=0)
    rm = jnp.stack([(((r - 1) >= 0)[:, None] * ones_c).reshape(1, H * W),
                    (((r + 1) < H)[:, None] * ones_c).reshape(1, H * W)], axis=0)
    return cm, rm


def _fused_kernel(x_ref, mask_ref, cmb_ref, rm_ref, wx_ref, wmid_ref, wpm_ref,
                  wo_ref, bxh_ref, bo_ref, out_ref,
                  xb_ref, inbr_ref, xz_ref, cstack_ref, u_ref,
                  *, T, C, hidden, W, HW, F, FHW):
    shifts = [dh * W + dw for dh in (-1, 0, 1) for dw in (-1, 0, 1)]

    for f in range(F):
        xb_ref[:, f * HW:(f + 1) * HW] = (
            x_ref[0, f * C:(f + 1) * C, :].astype(jnp.bfloat16))

    xv = xb_ref[...]
    for tap, s in enumerate(shifts):
        if s == 0:
            inbr_ref[tap * C:(tap + 1) * C, :] = xv
        else:
            inbr_ref[tap * C:(tap + 1) * C, :] = (
                pltpu.roll(xv, (-s) % FHW, axis=1) * mask_ref[tap])
    xz_ref[...] = (
        jnp.dot(wx_ref[...], inbr_ref[...], preferred_element_type=jnp.float32)
        + bxh_ref[...])

    def emit(h_bf, t):
        y = (jnp.dot(wo_ref[...], h_bf, preferred_element_type=jnp.float32)
             + bo_ref[...])
        for f in range(F):
            out_ref[0, f, t] = y[:, f * HW:(f + 1) * HW]

    h = jnp.tanh(xz_ref[...])
    emit(h.astype(jnp.bfloat16), 0)

    for t in range(1, T):
        cstack_ref[0:hidden, :] = (
            pltpu.roll(h, 1, axis=1).astype(jnp.bfloat16) * cmb_ref[0])
        cstack_ref[hidden:2 * hidden, :] = h.astype(jnp.bfloat16)
        cstack_ref[2 * hidden:3 * hidden, :] = (
            pltpu.roll(h, FHW - 1, axis=1).astype(jnp.bfloat16) * cmb_ref[1])
        u_ref[...] = jnp.dot(wpm_ref[...], cstack_ref[...],
                             preferred_element_type=jnp.float32)
        conv = (jnp.dot(wmid_ref[...], cstack_ref[...],
                        preferred_element_type=jnp.float32)
                + pltpu.roll(u_ref[0:hidden, :], W, axis=1) * rm_ref[0]
                + pltpu.roll(u_ref[hidden:2 * hidden, :], FHW - W,
                             axis=1) * rm_ref[1])
        h = jnp.tanh(xz_ref[...] + conv)
        emit(h.astype(jnp.bfloat16), t)


def kernel(seq_tensor, wx_t, wh_t, wo_t, bxh, bo):
    batch, seqlen, H, W, C = seq_tensor.shape
    N, HW = batch * seqlen, H * W
    T = 5
    hidden = wh_t.shape[1]

    F = max(1, min(2, N))
    while N % F != 0:
        F -= 1
    G = N // F
    FHW = F * HW

    x = seq_tensor.reshape(G, F * C, HW)

    wx = wx_t.astype(jnp.bfloat16)
    wg = (wh_t.reshape(3, 3, hidden, hidden)
          .transpose(0, 2, 1, 3)
          .reshape(3 * hidden, 3 * hidden).astype(jnp.bfloat16))
    wmid = wg[hidden:2 * hidden, :]
    wpm = jnp.concatenate([wg[0:hidden, :], wg[2 * hidden:3 * hidden, :]],
                          axis=0)
    masks = jnp.tile(_border_masks(H, W), (1, 1, F)).astype(jnp.bfloat16)
    cm, rm = _col_row_masks(H, W)
    cm = jnp.tile(cm, (1, 1, F)).astype(jnp.bfloat16)
    rm = jnp.tile(rm, (1, 1, F)).astype(jnp.float32)

    body = functools.partial(_fused_kernel, T=T, C=C, hidden=hidden,
                             W=W, HW=HW, F=F, FHW=FHW)
    y = pl.pallas_call(
        body,
        out_shape=jax.ShapeDtypeStruct((G, F, T, C, HW), jnp.float32),
        grid=(G,),
        in_specs=[
            pl.BlockSpec((1, F * C, HW), lambda g: (g, 0, 0)),
            pl.BlockSpec((9, 1, FHW), lambda g: (0, 0, 0)),
            pl.BlockSpec((2, 1, FHW), lambda g: (0, 0, 0)),
            pl.BlockSpec((2, 1, FHW), lambda g: (0, 0, 0)),
            pl.BlockSpec((hidden, 9 * C), lambda g: (0, 0)),
            pl.BlockSpec((hidden, 3 * hidden), lambda g: (0, 0)),
            pl.BlockSpec((2 * hidden, 3 * hidden), lambda g: (0, 0)),
            pl.BlockSpec((C, hidden), lambda g: (0, 0)),
            pl.BlockSpec((hidden, 1), lambda g: (0, 0)),
            pl.BlockSpec((C, 1), lambda g: (0, 0)),
        ],
        out_specs=pl.BlockSpec((1, F, T, C, HW), lambda g: (g, 0, 0, 0, 0)),
        scratch_shapes=[
            pltpu.VMEM((C, FHW), jnp.bfloat16),
            pltpu.VMEM((9 * C, FHW), jnp.bfloat16),
            pltpu.VMEM((hidden, FHW), jnp.float32),
            pltpu.VMEM((3 * hidden, FHW), jnp.bfloat16),
            pltpu.VMEM((2 * hidden, FHW), jnp.float32),
        ],
        compiler_params=pltpu.CompilerParams(
            dimension_semantics=("parallel",),
            vmem_limit_bytes=40 * 1024 * 1024),
    )(x, masks, cm, rm, wx, wmid, wpm, wo_t, bxh, bo)

    return y.reshape(N, T, C, H, W)
